# baseline (speedup 1.0000x reference)
import functools

import jax
import jax.numpy as jnp
from jax import lax
from jax.experimental import pallas as pl
from jax.experimental.pallas import tpu as pltpu
from jax.experimental.pallas import tpu_sc as plsc

N = 10000
E = 320000
F = 128
HID = 64
CLS = 32

NC = 2
NS = 16
NW = NC * NS
EPW = E // NW
B = 200
K = EPW // B
NPAD = 10240
PAD = NPAD - N
R = NPAD // NS
NB = 2

_MESH = plsc.VectorSubcoreMesh(core_axis_name="c", subcore_axis_name="s")
_SC_PARAMS = pltpu.CompilerParams(use_tc_tiling_on_sc=False)
_SC_PARAMS_NL = pltpu.CompilerParams(use_tc_tiling_on_sc=False,
                                     needs_layout_passes=False)


def _rsqrt16(d):
    u = plsc.bitcast(d, jnp.int32)
    u = jnp.int32(0x5F3759DF) - lax.shift_right_logical(u, 1)
    y = plsc.bitcast(u, jnp.float32)
    for _ in range(3):
        y = y * (1.5 - 0.5 * d * y * y)
    return y


@functools.partial(
    pl.kernel,
    out_type=(
        jax.ShapeDtypeStruct((NC, NPAD, HID), jnp.float32),
        jax.ShapeDtypeStruct((NPAD, HID), jnp.float32),
        jax.ShapeDtypeStruct((NPAD,), jnp.float32),
    ),
    mesh=_MESH,
    compiler_params=_SC_PARAMS_NL,
    scratch_types=[
        pltpu.VMEM((K, B), jnp.int32),
        pltpu.VMEM((K, B), jnp.int32),
        [pltpu.VMEM((B, HID), jnp.float32)] * NB,
        pltpu.VMEM((R, HID), jnp.float32),
        pltpu.VMEM((256,), jnp.float32),
        pltpu.VMEM((R,), jnp.float32),
        pltpu.VMEM((R,), jnp.float32),
        pltpu.VMEM_SHARED((NPAD, HID), jnp.float32),
        pltpu.VMEM_SHARED((NPAD,), jnp.float32),
        [pltpu.SemaphoreType.DMA] * NB,
    ],
)
def _deg_agg_kernel(h_hbm, src_hbm, dst_hbm, p_hbm, g_hbm, dinv_hbm,
                    src_v, dst_v, rows, hrows_v, ones_v, deg_v, dinv_v,
                    acc_sh, deg_sh, sems):
    c = lax.axis_index("c")
    s = lax.axis_index("s")
    wid = c * NS + s

    def fill(i, carry):
        ones_v[pl.ds(i * 16, 16)] = jnp.ones((16,), jnp.float32)
        return carry

    lax.fori_loop(0, 16, fill, 0)

    def seed(j, carry):
        pltpu.sync_copy(ones_v.at[pl.ds(0, 128)],
                        deg_sh.at[pl.ds(s * R + j * 128, 128)])
        return carry

    lax.fori_loop(0, R // 128, seed, 0)
    plsc.subcore_barrier()

    for half in range(NC):
        pltpu.sync_copy(dst_hbm.at[half * NS + s], dst_v)

        def dbody(j, carry):
            pltpu.sync_copy(ones_v.at[pl.ds(0, B)], deg_sh.at[dst_v.at[j]],
                            add=True)
            return carry

        lax.fori_loop(0, K, dbody, 0)
    plsc.subcore_barrier()

    pltpu.sync_copy(deg_sh.at[pl.ds(s * R, R)], deg_v)

    def rbody(i, carry):
        dinv_v[pl.ds(i * 16, 16)] = _rsqrt16(deg_v[pl.ds(i * 16, 16)])
        return carry

    lax.fori_loop(0, R // 16, rbody, 0)
    pltpu.sync_copy(dinv_v, dinv_hbm.at[pl.ds(s * R, R)])

    pltpu.sync_copy(h_hbm.at[pl.ds(s * R, R)], hrows_v)

    def sbody(i, carry):
        dvec = dinv_v[pl.ds(i * 16, 16)]
        for r16 in range(16):
            r = i * 16 + r16
            d = dvec[r16]
            for cc in range(HID // 16):
                sl = pl.ds(cc * 16, 16)
                hrows_v[r, sl] = hrows_v[r, sl] * d
        return carry

    lax.fori_loop(0, R // 16, sbody, 0)
    pltpu.sync_copy(hrows_v, g_hbm.at[pl.ds(s * R, R)])
    pltpu.sync_copy(hrows_v, acc_sh.at[pl.ds(s * R, R)])
    pltpu.sync_copy(src_hbm.at[wid], src_v)
    pltpu.sync_copy(dst_hbm.at[wid], dst_v)
    plsc.subcore_barrier()

    for b in range(NB):
        pltpu.async_copy(g_hbm.at[src_v.at[b]], rows[b], sems[b])

    def body(i, carry):
        for b in range(NB):
            j = NB * i + b
            pltpu.make_async_copy(g_hbm.at[src_v.at[j]], rows[b],
                                  sems[b]).wait()
            pltpu.sync_copy(rows[b], acc_sh.at[dst_v.at[j]], add=True)
            jn = jnp.minimum(j + NB, K - 1)
            pltpu.async_copy(g_hbm.at[src_v.at[jn]], rows[b], sems[b])
        return carry

    lax.fori_loop(0, K // NB, body, 0)
    for b in range(NB):
        pltpu.make_async_copy(g_hbm.at[src_v.at[K - 1]], rows[b],
                              sems[b]).wait()
    plsc.subcore_barrier()
    pltpu.sync_copy(acc_sh.at[pl.ds(s * R, R)],
                    p_hbm.at[c].at[pl.ds(s * R, R)])


@functools.partial(
    pl.kernel,
    out_type=jax.ShapeDtypeStruct((NC, NPAD, CLS), jnp.float32),
    mesh=_MESH,
    compiler_params=_SC_PARAMS,
    scratch_types=[
        pltpu.VMEM((K, B), jnp.int32),
        pltpu.VMEM((K, B), jnp.int32),
        [pltpu.VMEM((B, CLS), jnp.float32)] * NB,
        pltpu.VMEM_SHARED((NPAD, CLS), jnp.float32),
        [pltpu.SemaphoreType.DMA] * NB,
    ],
)
def _agg_c(g_hbm, src_hbm, dst_hbm, out_hbm, src_v, dst_v, rows,
           acc_sh, sems):
    c = lax.axis_index("c")
    s = lax.axis_index("s")
    wid = c * NS + s
    pltpu.sync_copy(g_hbm.at[pl.ds(s * R, R)], acc_sh.at[pl.ds(s * R, R)])
    pltpu.sync_copy(src_hbm.at[wid], src_v)
    pltpu.sync_copy(dst_hbm.at[wid], dst_v)
    plsc.subcore_barrier()

    for b in range(NB):
        pltpu.async_copy(g_hbm.at[src_v.at[b]], rows[b], sems[b])

    def body(i, carry):
        for b in range(NB):
            j = NB * i + b
            pltpu.make_async_copy(g_hbm.at[src_v.at[j]], rows[b],
                                  sems[b]).wait()
            pltpu.sync_copy(rows[b], acc_sh.at[dst_v.at[j]], add=True)
            jn = jnp.minimum(j + NB, K - 1)
            pltpu.async_copy(g_hbm.at[src_v.at[jn]], rows[b], sems[b])
        return carry

    lax.fori_loop(0, K // NB, body, 0)
    for b in range(NB):
        pltpu.make_async_copy(g_hbm.at[src_v.at[K - 1]], rows[b],
                              sems[b]).wait()
    plsc.subcore_barrier()
    pltpu.sync_copy(acc_sh.at[pl.ds(s * R, R)],
                    out_hbm.at[c].at[pl.ds(s * R, R)])


def _tc1_body(x_ref, w1_ref, h_ref):
    h = jnp.dot(x_ref[...], w1_ref[...], preferred_element_type=jnp.float32)
    h_ref[0:N, :] = h
    h_ref[pl.ds(N, PAD), :] = jnp.zeros((PAD, HID), jnp.float32)


def _tc2_body(p_ref, g1_ref, dinv_ref, w2_ref, b1_ref, g2_ref):
    dinv = dinv_ref[0:N][:, None]
    agg = p_ref[0, 0:N, :] + p_ref[1, 0:N, :] - g1_ref[0:N, :]
    h1 = jnp.maximum(agg * dinv + b1_ref[...], 0.0)
    g2 = jnp.dot(h1, w2_ref[...], preferred_element_type=jnp.float32)
    g2_ref[0:N, :] = g2 * dinv


def _tc3_body(q_ref, g2_ref, dinv_ref, b2_ref, out_ref):
    out_ref[...] = (q_ref[0, 0:N, :] + q_ref[1, 0:N, :] - g2_ref[0:N, :]) \
        * dinv_ref[0:N][:, None] + b2_ref[...]


_tc1 = pl.pallas_call(
    _tc1_body,
    out_shape=jax.ShapeDtypeStruct((NPAD, HID), jnp.float32),
)

_tc2 = pl.pallas_call(
    _tc2_body,
    out_shape=jax.ShapeDtypeStruct((NPAD, CLS), jnp.float32),
)

_tc3 = pl.pallas_call(
    _tc3_body,
    out_shape=jax.ShapeDtypeStruct((N, CLS), jnp.float32),
)


@jax.jit
def kernel(x, edges, W1, b1, W2, b2):
    src = edges[0].astype(jnp.int32).reshape(NW, K, B)
    dst = edges[1].astype(jnp.int32).reshape(NW, K, B)
    h = _tc1(x, W1)
    p, g1p, dinv = _deg_agg_kernel(h, src, dst)
    g2p = _tc2(p, g1p, dinv, W2, b1.reshape(1, HID))
    q = _agg_c(g2p, src, dst)
    return _tc3(q, g2p, dinv, b2.reshape(1, CLS))


# speedup vs baseline: 1.0850x; 1.0850x over previous
import functools

import jax
import jax.numpy as jnp
from jax import lax
from jax.experimental import pallas as pl
from jax.experimental.pallas import tpu as pltpu
from jax.experimental.pallas import tpu_sc as plsc

N = 10000
E = 320000
F = 128
HID = 64
CLS = 32

NC = 2
NS = 16
NW = NC * NS
EPW = E // NW
B = 250
K = EPW // B
NPAD = 10240
PAD = NPAD - N
R = NPAD // NS
DEGW = 16

_MESH = plsc.VectorSubcoreMesh(core_axis_name="c", subcore_axis_name="s")
_SC_PARAMS = pltpu.CompilerParams(use_tc_tiling_on_sc=False)


@functools.partial(
    pl.kernel,
    out_type=jax.ShapeDtypeStruct((NC, NPAD), jnp.float32),
    mesh=_MESH,
    compiler_params=_SC_PARAMS,
    scratch_types=[
        pltpu.VMEM((K, B), jnp.int32),
        pltpu.VMEM((256,), jnp.float32),
        pltpu.VMEM((R,), jnp.float32),
        pltpu.VMEM_SHARED((NPAD,), jnp.float32),
    ],
)
def _deg_kernel(dst_hbm, out_hbm, dst_v, ones_v, bounce_v, acc_sh):
    c = lax.axis_index("c")
    s = lax.axis_index("s")
    wid = c * NS + s

    def fill(i, carry):
        ones_v[pl.ds(i * 16, 16)] = jnp.ones((16,), jnp.float32)
        return carry

    lax.fori_loop(0, 16, fill, 0)

    def seed(j, carry):
        pltpu.sync_copy(ones_v.at[pl.ds(0, 128)],
                        acc_sh.at[pl.ds(s * R + j * 128, 128)])
        return carry

    lax.fori_loop(0, R // 128, seed, 0)
    pltpu.sync_copy(dst_hbm.at[wid], dst_v)
    plsc.subcore_barrier()

    def body(j, carry):
        pltpu.sync_copy(ones_v.at[pl.ds(0, B)], acc_sh.at[dst_v.at[j]],
                        add=True)
        return carry

    lax.fori_loop(0, K, body, 0)
    plsc.subcore_barrier()
    pltpu.sync_copy(acc_sh.at[pl.ds(s * R, R)],
                    out_hbm.at[c].at[pl.ds(s * R, R)])


NB = 2


def _make_agg(D):
    @functools.partial(
        pl.kernel,
        out_type=jax.ShapeDtypeStruct((NC, NPAD, D), jnp.float32),
        mesh=_MESH,
        compiler_params=_SC_PARAMS,
        scratch_types=[
            pltpu.VMEM((K, B), jnp.int32),
            pltpu.VMEM((K, B), jnp.int32),
            [pltpu.VMEM((B, D), jnp.float32)] * NB,
            pltpu.VMEM((R, D), jnp.float32),
            pltpu.VMEM_SHARED((NPAD, D), jnp.float32),
            [pltpu.SemaphoreType.DMA] * NB,
        ],
    )
    def agg(g_hbm, src_hbm, dst_hbm, out_hbm, src_v, dst_v, rows, bounce_v,
            acc_sh, sems):
        c = lax.axis_index("c")
        s = lax.axis_index("s")
        wid = c * NS + s
        pltpu.sync_copy(g_hbm.at[pl.ds(s * R, R)], acc_sh.at[pl.ds(s * R, R)])
        pltpu.sync_copy(src_hbm.at[wid], src_v)
        pltpu.sync_copy(dst_hbm.at[wid], dst_v)
        plsc.subcore_barrier()

        for b in range(NB):
            pltpu.async_copy(g_hbm.at[src_v.at[b]], rows[b], sems[b])

        def body(i, carry):
            for b in range(NB):
                j = NB * i + b
                pltpu.make_async_copy(g_hbm.at[src_v.at[j]], rows[b],
                                      sems[b]).wait()
                pltpu.sync_copy(rows[b], acc_sh.at[dst_v.at[j]], add=True)
                jn = jnp.minimum(j + NB, K - 1)
                pltpu.async_copy(g_hbm.at[src_v.at[jn]], rows[b], sems[b])
            return carry

        lax.fori_loop(0, K // NB, body, 0)
        for b in range(NB):
            pltpu.make_async_copy(g_hbm.at[src_v.at[K - 1]], rows[b],
                                  sems[b]).wait()
        plsc.subcore_barrier()
        pltpu.sync_copy(acc_sh.at[pl.ds(s * R, R)],
                        out_hbm.at[c].at[pl.ds(s * R, R)])

    return agg


_agg_h = _make_agg(HID)
_agg_c = _make_agg(CLS)


def _tc1_body(x_ref, w1_ref, degp_ref, g1_ref, dinv_ref):
    deg = degp_ref[0, 0:N] + degp_ref[1, 0:N] - 1.0
    dinv = jax.lax.rsqrt(deg)[:, None]
    h = jnp.dot(x_ref[...], w1_ref[...], preferred_element_type=jnp.float32)
    g1_ref[0:N, :] = h * dinv
    dinv_ref[...] = dinv


def _tc2_body(p_ref, g1_ref, dinv_ref, w2_ref, b1_ref, g2_ref):
    agg = p_ref[0, 0:N, :] + p_ref[1, 0:N, :] - g1_ref[0:N, :]
    h1 = jnp.maximum(agg * dinv_ref[...] + b1_ref[...], 0.0)
    g2 = jnp.dot(h1, w2_ref[...], preferred_element_type=jnp.float32)
    g2_ref[0:N, :] = g2 * dinv_ref[...]


def _tc3_body(q_ref, g2_ref, dinv_ref, b2_ref, out_ref):
    out_ref[...] = (q_ref[0, 0:N, :] + q_ref[1, 0:N, :] - g2_ref[0:N, :]) \
        * dinv_ref[...] + b2_ref[...]


_tc1 = pl.pallas_call(
    _tc1_body,
    out_shape=(
        jax.ShapeDtypeStruct((NPAD, HID), jnp.float32),
        jax.ShapeDtypeStruct((N, 1), jnp.float32),
    ),
)

_tc2 = pl.pallas_call(
    _tc2_body,
    out_shape=jax.ShapeDtypeStruct((NPAD, CLS), jnp.float32),
)

_tc3 = pl.pallas_call(
    _tc3_body,
    out_shape=jax.ShapeDtypeStruct((N, CLS), jnp.float32),
)


@jax.jit
def kernel(x, edges, W1, b1, W2, b2):
    src = edges[0].astype(jnp.int32).reshape(NW, K, B)
    dst = edges[1].astype(jnp.int32).reshape(NW, K, B)
    degp = _deg_kernel(dst)
    g1p, dinv = _tc1(x, W1, degp)
    p = _agg_h(g1p, src, dst)
    g2p = _tc2(p, g1p, dinv, W2, b1.reshape(1, HID))
    q = _agg_c(g2p, src, dst)
    return _tc3(q, g2p, dinv, b2.reshape(1, CLS))


# speedup vs baseline: 1.0927x; 1.0071x over previous
import functools

import jax
import jax.numpy as jnp
from jax import lax
from jax.experimental import pallas as pl
from jax.experimental.pallas import tpu as pltpu
from jax.experimental.pallas import tpu_sc as plsc

N = 10000
E = 320000
F = 128
HID = 64
CLS = 32

NC = 2
NS = 16
NW = NC * NS
EPW = E // NW
B1 = 250
K1 = EPW // B1
B2 = 500
K2 = EPW // B2
NPAD = 10240
PAD = NPAD - N
R = NPAD // NS
DEGW = 16

_MESH = plsc.VectorSubcoreMesh(core_axis_name="c", subcore_axis_name="s")
_SC_PARAMS = pltpu.CompilerParams(use_tc_tiling_on_sc=False)


@functools.partial(
    pl.kernel,
    out_type=jax.ShapeDtypeStruct((NC, NPAD), jnp.float32),
    mesh=_MESH,
    compiler_params=_SC_PARAMS,
    scratch_types=[
        pltpu.VMEM((K2, B2), jnp.int32),
        pltpu.VMEM((512,), jnp.float32),
        pltpu.VMEM((R,), jnp.float32),
        pltpu.VMEM_SHARED((NPAD,), jnp.float32),
    ],
)
def _deg_kernel(dst_hbm, out_hbm, dst_v, ones_v, bounce_v, acc_sh):
    c = lax.axis_index("c")
    s = lax.axis_index("s")
    wid = c * NS + s

    def fill(i, carry):
        ones_v[pl.ds(i * 16, 16)] = jnp.ones((16,), jnp.float32)
        return carry

    lax.fori_loop(0, 32, fill, 0)

    def seed(j, carry):
        pltpu.sync_copy(ones_v.at[pl.ds(0, 128)],
                        acc_sh.at[pl.ds(s * R + j * 128, 128)])
        return carry

    lax.fori_loop(0, R // 128, seed, 0)
    pltpu.sync_copy(dst_hbm.at[wid], dst_v)
    plsc.subcore_barrier()

    def body(j, carry):
        pltpu.sync_copy(ones_v.at[pl.ds(0, B2)], acc_sh.at[dst_v.at[j]],
                        add=True)
        return carry

    lax.fori_loop(0, K2, body, 0)
    plsc.subcore_barrier()
    pltpu.sync_copy(acc_sh.at[pl.ds(s * R, R)],
                    out_hbm.at[c].at[pl.ds(s * R, R)])


NB = 2


def _make_agg(D, B, K):
    @functools.partial(
        pl.kernel,
        out_type=jax.ShapeDtypeStruct((NC, NPAD, D), jnp.float32),
        mesh=_MESH,
        compiler_params=_SC_PARAMS,
        scratch_types=[
            pltpu.VMEM((K, B), jnp.int32),
            pltpu.VMEM((K, B), jnp.int32),
            [pltpu.VMEM((B, D), jnp.float32)] * NB,
            pltpu.VMEM((R, D), jnp.float32),
            pltpu.VMEM_SHARED((NPAD, D), jnp.float32),
            [pltpu.SemaphoreType.DMA] * NB,
        ],
    )
    def agg(g_hbm, src_hbm, dst_hbm, out_hbm, src_v, dst_v, rows, bounce_v,
            acc_sh, sems):
        c = lax.axis_index("c")
        s = lax.axis_index("s")
        wid = c * NS + s
        pltpu.sync_copy(g_hbm.at[pl.ds(s * R, R)], acc_sh.at[pl.ds(s * R, R)])
        pltpu.sync_copy(src_hbm.at[wid], src_v)
        pltpu.sync_copy(dst_hbm.at[wid], dst_v)
        plsc.subcore_barrier()

        for b in range(NB):
            pltpu.async_copy(g_hbm.at[src_v.at[b]], rows[b], sems[b])

        def body(i, carry):
            for b in range(NB):
                j = NB * i + b
                pltpu.make_async_copy(g_hbm.at[src_v.at[j]], rows[b],
                                      sems[b]).wait()
                pltpu.sync_copy(rows[b], acc_sh.at[dst_v.at[j]], add=True)
                jn = jnp.minimum(j + NB, K - 1)
                pltpu.async_copy(g_hbm.at[src_v.at[jn]], rows[b], sems[b])
            return carry

        lax.fori_loop(0, K // NB, body, 0)
        for b in range(NB):
            pltpu.make_async_copy(g_hbm.at[src_v.at[K - 1]], rows[b],
                                  sems[b]).wait()
        plsc.subcore_barrier()
        pltpu.sync_copy(acc_sh.at[pl.ds(s * R, R)],
                        out_hbm.at[c].at[pl.ds(s * R, R)])

    return agg


_agg_h = _make_agg(HID, B1, K1)
_agg_c = _make_agg(CLS, B2, K2)


def _tc1_body(x_ref, w1_ref, degp_ref, g1_ref, dinv_ref):
    deg = degp_ref[0, 0:N] + degp_ref[1, 0:N] - 1.0
    dinv = jax.lax.rsqrt(deg)[:, None]
    h = jnp.dot(x_ref[...], w1_ref[...], preferred_element_type=jnp.float32)
    g1_ref[0:N, :] = h * dinv
    dinv_ref[...] = dinv


def _tc2_body(p_ref, g1_ref, dinv_ref, w2_ref, b1_ref, g2_ref):
    agg = p_ref[0, 0:N, :] + p_ref[1, 0:N, :] - g1_ref[0:N, :]
    h1 = jnp.maximum(agg * dinv_ref[...] + b1_ref[...], 0.0)
    g2 = jnp.dot(h1, w2_ref[...], preferred_element_type=jnp.float32)
    g2_ref[0:N, :] = g2 * dinv_ref[...]


def _tc3_body(q_ref, g2_ref, dinv_ref, b2_ref, out_ref):
    out_ref[...] = (q_ref[0, 0:N, :] + q_ref[1, 0:N, :] - g2_ref[0:N, :]) \
        * dinv_ref[...] + b2_ref[...]


_tc1 = pl.pallas_call(
    _tc1_body,
    out_shape=(
        jax.ShapeDtypeStruct((NPAD, HID), jnp.float32),
        jax.ShapeDtypeStruct((N, 1), jnp.float32),
    ),
)

_tc2 = pl.pallas_call(
    _tc2_body,
    out_shape=jax.ShapeDtypeStruct((NPAD, CLS), jnp.float32),
)

_tc3 = pl.pallas_call(
    _tc3_body,
    out_shape=jax.ShapeDtypeStruct((N, CLS), jnp.float32),
)


@jax.jit
def kernel(x, edges, W1, b1, W2, b2):
    src = edges[0].astype(jnp.int32)
    dst = edges[1].astype(jnp.int32)
    src1 = src.reshape(NW, K1, B1)
    dst1 = dst.reshape(NW, K1, B1)
    src2 = src.reshape(NW, K2, B2)
    dst2 = dst.reshape(NW, K2, B2)
    degp = _deg_kernel(dst2)
    g1p, dinv = _tc1(x, W1, degp)
    p = _agg_h(g1p, src1, dst1)
    g2p = _tc2(p, g1p, dinv, W2, b1.reshape(1, HID))
    q = _agg_c(g2p, src2, dst2)
    return _tc3(q, g2p, dinv, b2.reshape(1, CLS))
